# baseline (speedup 1.0000x reference)
import functools

import jax
import jax.numpy as jnp
from jax import lax
from jax.experimental import pallas as pl
from jax.experimental.pallas import tpu as pltpu
from jax.experimental.pallas import tpu_sc as plsc

NUM_ROWS = 4096 * 200
DIM = 64
IDXV = 128
NUM_IDXV = NUM_ROWS // IDXV
NW = 32
VPW = NUM_IDXV // NW
G = 4
CHUNKS = VPW // G


def _make_kernel():
    info = plsc.get_sparse_core_info()
    nc = info.num_cores
    mesh = plsc.VectorSubcoreMesh(core_axis_name="c", subcore_axis_name="s")

    @functools.partial(
        pl.kernel,
        out_type=jax.ShapeDtypeStruct((NUM_ROWS, DIM), jnp.float32),
        mesh=mesh,
        scratch_types=[
            pltpu.VMEM((2, G, IDXV), jnp.int32),
            pltpu.VMEM((2, G * IDXV, DIM), jnp.float32),
            pltpu.SemaphoreType.DMA,
            pltpu.SemaphoreType.DMA,
        ],
        compiler_params=pltpu.CompilerParams(use_tc_tiling_on_sc=False),
    )
    def body(idx_hbm, table_hbm, out_hbm, idx_v, rows_v, gsem, osem):
        wid = lax.axis_index("s") * nc + lax.axis_index("c")
        base = wid * VPW

        def fire(slot, chunk):
            r0 = base + chunk * G
            pltpu.sync_copy(idx_hbm.at[pl.ds(r0, G)], idx_v.at[slot])
            for j in range(G):
                pltpu.async_copy(
                    table_hbm.at[idx_v.at[slot].at[j]],
                    rows_v.at[slot].at[pl.ds(j * IDXV, IDXV)],
                    gsem,
                )

        def drain_and_store(slot, chunk):
            for j in range(G):
                pltpu.make_async_copy(
                    table_hbm.at[idx_v.at[slot].at[j]],
                    rows_v.at[slot].at[pl.ds(j * IDXV, IDXV)],
                    gsem,
                ).wait()
            r0 = base + chunk * G
            pltpu.async_copy(
                rows_v.at[slot], out_hbm.at[pl.ds(r0 * IDXV, G * IDXV)], osem
            )

        def store_wait(slot, chunk):
            r0 = base + chunk * G
            pltpu.make_async_copy(
                rows_v.at[slot], out_hbm.at[pl.ds(r0 * IDXV, G * IDXV)], osem
            ).wait()

        fire(0, 0)

        @pl.loop(0, CHUNKS - 1)
        def _(i):
            slot = lax.rem(i, 2)
            nxt = 1 - slot
            fire(nxt, i + 1)
            drain_and_store(slot, i)
            store_wait(slot, i)

        last = (CHUNKS - 1) % 2
        drain_and_store(last, CHUNKS - 1)
        store_wait(last, CHUNKS - 1)

    return body


_gather = _make_kernel()


@jax.jit
def kernel(x, weight):
    idx = x.reshape(NUM_IDXV, IDXV).astype(jnp.int32)
    out = _gather(idx, weight)
    return out.reshape(x.shape[0], x.shape[1], DIM)


# speedup vs baseline: 1.0085x; 1.0085x over previous
import functools

import jax
import jax.numpy as jnp
from jax import lax
from jax.experimental import pallas as pl
from jax.experimental.pallas import tpu as pltpu
from jax.experimental.pallas import tpu_sc as plsc

NUM_ROWS = 4096 * 200
DIM = 64
IDXV = 128
NUM_IDXV = NUM_ROWS // IDXV
NW = 32
VPW = NUM_IDXV // NW
G = 5
CHUNK = G * IDXV
CHUNKS = VPW // G


def _make_kernel():
    info = plsc.get_sparse_core_info()
    nc = info.num_cores
    mesh = plsc.VectorSubcoreMesh(core_axis_name="c", subcore_axis_name="s")

    @functools.partial(
        pl.kernel,
        out_type=jax.ShapeDtypeStruct((NUM_ROWS, DIM), jnp.float32),
        mesh=mesh,
        scratch_types=[
            pltpu.VMEM((VPW, IDXV), jnp.int32),
            pltpu.VMEM((2, CHUNK, DIM), jnp.float32),
            pltpu.SemaphoreType.DMA,
            pltpu.SemaphoreType.DMA,
            pltpu.SemaphoreType.DMA,
            pltpu.SemaphoreType.DMA,
            pltpu.SemaphoreType.DMA,
        ],
        compiler_params=pltpu.CompilerParams(use_tc_tiling_on_sc=False),
    )
    def body(idx_hbm, table_hbm, out_hbm, idx_v, rows_v, isem, g0, g1, s0, s1):
        wid = lax.axis_index("s") * nc + lax.axis_index("c")
        base = wid * VPW
        gsem = (g0, g1)
        ssem = (s0, s1)

        pltpu.async_copy(idx_hbm.at[pl.ds(base, VPW)], idx_v, isem).wait()

        def fire(slot, chunk):
            for j in range(G):
                pltpu.async_copy(
                    table_hbm.at[idx_v.at[chunk * G + j]],
                    rows_v.at[slot].at[pl.ds(j * IDXV, IDXV)],
                    gsem[slot],
                )

        def wait_gathers(slot, chunk):
            for j in range(G):
                pltpu.make_async_copy(
                    table_hbm.at[idx_v.at[chunk * G + j]],
                    rows_v.at[slot].at[pl.ds(j * IDXV, IDXV)],
                    gsem[slot],
                ).wait()

        def store(slot, chunk):
            pltpu.async_copy(
                rows_v.at[slot],
                out_hbm.at[pl.ds((base + chunk * G) * IDXV, CHUNK)],
                ssem[slot],
            )

        def wait_store(slot, chunk):
            pltpu.make_async_copy(
                rows_v.at[slot],
                out_hbm.at[pl.ds((base + chunk * G) * IDXV, CHUNK)],
                ssem[slot],
            ).wait()

        fire(0, 0)
        fire(1, 1)
        wait_gathers(0, 0)
        store(0, 0)

        @pl.loop(0, CHUNKS // 2 - 1)
        def _(k):
            c1 = 2 * k + 1
            wait_store(0, c1 - 1)
            fire(0, c1 + 1)
            wait_gathers(1, c1)
            store(1, c1)

            c2 = 2 * k + 2
            wait_store(1, c2 - 1)
            fire(1, c2 + 1)
            wait_gathers(0, c2)
            store(0, c2)

        wait_store(0, CHUNKS - 2)
        wait_gathers(1, CHUNKS - 1)
        store(1, CHUNKS - 1)
        wait_store(1, CHUNKS - 1)

    return body


_gather = _make_kernel()


@jax.jit
def kernel(x, weight):
    idx = x.reshape(NUM_IDXV, IDXV).astype(jnp.int32)
    out = _gather(idx, weight)
    return out.reshape(x.shape[0], x.shape[1], DIM)


# speedup vs baseline: 1.2296x; 1.2193x over previous
import functools

import jax
import jax.numpy as jnp
from jax import lax
from jax.experimental import pallas as pl
from jax.experimental.pallas import tpu as pltpu
from jax.experimental.pallas import tpu_sc as plsc

B, S = 4096, 200
NUM_ROWS = B * S
DIM = 64
PDIM = 128
NW = 32
RPW = NUM_ROWS // NW
CHUNK = 256
CHUNKS = RPW // CHUNK
NG = CHUNK // 128


def _make_kernel():
    info = plsc.get_sparse_core_info()
    nc = info.num_cores
    mesh = plsc.VectorSubcoreMesh(core_axis_name="c", subcore_axis_name="s")

    @functools.partial(
        pl.kernel,
        out_type=jax.ShapeDtypeStruct((NUM_ROWS, PDIM), jnp.float32),
        mesh=mesh,
        scratch_types=[
            pltpu.VMEM((RPW,), jnp.int32),
            pltpu.VMEM((2, CHUNK, PDIM), jnp.float32),
            pltpu.SemaphoreType.DMA,
            pltpu.SemaphoreType.DMA,
            pltpu.SemaphoreType.DMA,
            pltpu.SemaphoreType.DMA,
            pltpu.SemaphoreType.DMA,
        ],
        compiler_params=pltpu.CompilerParams(use_tc_tiling_on_sc=True),
    )
    def body(xf_hbm, table_hbm, out_hbm, idx_v, rows_v, isem, g0, g1, s0, s1):
        wid = lax.axis_index("s") * nc + lax.axis_index("c")
        base = wid * RPW
        gsem = (g0, g1)
        ssem = (s0, s1)

        pltpu.async_copy(xf_hbm.at[pl.ds(base, RPW)], idx_v, isem).wait()

        def g_pairs(slot, chunk):
            for j in range(NG):
                src = table_hbm.at[idx_v.at[pl.ds(chunk * CHUNK + j * 128, 128)]]
                dst = rows_v.at[slot].at[pl.ds(j * 128, 128)]
                yield src, dst

        def fire(slot, chunk):
            for src, dst in g_pairs(slot, chunk):
                pltpu.async_copy(src, dst, gsem[slot])

        def wait_gathers(slot, chunk):
            for src, dst in g_pairs(slot, chunk):
                pltpu.make_async_copy(src, dst, gsem[slot]).wait()

        def s_pair(slot, chunk):
            src = rows_v.at[slot]
            dst = out_hbm.at[pl.ds(base + chunk * CHUNK, CHUNK)]
            return src, dst

        def store(slot, chunk):
            src, dst = s_pair(slot, chunk)
            pltpu.async_copy(src, dst, ssem[slot])

        def wait_store(slot, chunk):
            src, dst = s_pair(slot, chunk)
            pltpu.make_async_copy(src, dst, ssem[slot]).wait()

        fire(0, 0)
        fire(1, 1)
        wait_gathers(0, 0)
        store(0, 0)

        @pl.loop(0, CHUNKS // 2 - 1)
        def _(k):
            c1 = 2 * k + 1
            wait_store(0, c1 - 1)
            fire(0, c1 + 1)
            wait_gathers(1, c1)
            store(1, c1)

            c2 = 2 * k + 2
            wait_store(1, c2 - 1)
            fire(1, c2 + 1)
            wait_gathers(0, c2)
            store(0, c2)

        wait_store(0, CHUNKS - 2)
        wait_gathers(1, CHUNKS - 1)
        store(1, CHUNKS - 1)
        wait_store(1, CHUNKS - 1)

    return body


_gather = _make_kernel()


@jax.jit
def kernel(x, weight):
    xf = x.reshape(-1).astype(jnp.int32)
    wp = jnp.pad(weight, ((0, 0), (0, PDIM - DIM)))
    out = _gather(xf, wp)
    return out[:, :DIM].reshape(B, S, DIM)


# speedup vs baseline: 1.2304x; 1.0006x over previous
import functools

import jax
import jax.numpy as jnp
from jax import lax
from jax.experimental import pallas as pl
from jax.experimental.pallas import tpu as pltpu
from jax.experimental.pallas import tpu_sc as plsc

B, S = 4096, 200
NUM_ROWS = B * S
DIM = 64
PDIM = 128
NW = 32
RPW = NUM_ROWS // NW
CHUNK = 320
CHUNKS = RPW // CHUNK
NG = CHUNK // 128


def _make_kernel():
    info = plsc.get_sparse_core_info()
    nc = info.num_cores
    mesh = plsc.VectorSubcoreMesh(core_axis_name="c", subcore_axis_name="s")

    @functools.partial(
        pl.kernel,
        out_type=jax.ShapeDtypeStruct((NUM_ROWS, PDIM), jnp.float32),
        mesh=mesh,
        scratch_types=[
            pltpu.VMEM((RPW,), jnp.int32),
            pltpu.VMEM((2, CHUNK, PDIM), jnp.float32),
            pltpu.SemaphoreType.DMA,
            pltpu.SemaphoreType.DMA,
            pltpu.SemaphoreType.DMA,
            pltpu.SemaphoreType.DMA,
            pltpu.SemaphoreType.DMA,
        ],
        compiler_params=pltpu.CompilerParams(use_tc_tiling_on_sc=True),
    )
    def body(xf_hbm, table_hbm, out_hbm, idx_v, rows_v, isem, g0, g1, s0, s1):
        wid = lax.axis_index("s") * nc + lax.axis_index("c")
        base = wid * RPW
        gsem = (g0, g1)
        ssem = (s0, s1)

        pltpu.async_copy(xf_hbm.at[pl.ds(base, RPW)], idx_v, isem).wait()

        def g_pairs(slot, chunk):
            off = 0
            for glen in (128, 128, 64):
                src = table_hbm.at[idx_v.at[pl.ds(chunk * CHUNK + off, glen)]]
                dst = rows_v.at[slot].at[pl.ds(off, glen)]
                off += glen
                yield src, dst

        def fire(slot, chunk):
            for src, dst in g_pairs(slot, chunk):
                pltpu.async_copy(src, dst, gsem[slot])

        def wait_gathers(slot, chunk):
            for src, dst in g_pairs(slot, chunk):
                pltpu.make_async_copy(src, dst, gsem[slot]).wait()

        def s_pair(slot, chunk):
            src = rows_v.at[slot]
            dst = out_hbm.at[pl.ds(base + chunk * CHUNK, CHUNK)]
            return src, dst

        def store(slot, chunk):
            src, dst = s_pair(slot, chunk)
            pltpu.async_copy(src, dst, ssem[slot])

        def wait_store(slot, chunk):
            src, dst = s_pair(slot, chunk)
            pltpu.make_async_copy(src, dst, ssem[slot]).wait()

        fire(0, 0)
        fire(1, 1)
        wait_gathers(0, 0)
        store(0, 0)

        @pl.loop(0, CHUNKS // 2 - 1)
        def _(k):
            c1 = 2 * k + 1
            wait_store(0, c1 - 1)
            fire(0, c1 + 1)
            wait_gathers(1, c1)
            store(1, c1)

            c2 = 2 * k + 2
            wait_store(1, c2 - 1)
            fire(1, c2 + 1)
            wait_gathers(0, c2)
            store(0, c2)

        wait_store(0, CHUNKS - 2)
        wait_gathers(1, CHUNKS - 1)
        store(1, CHUNKS - 1)
        wait_store(1, CHUNKS - 1)

    return body


_gather = _make_kernel()


@jax.jit
def kernel(x, weight):
    xf = x.reshape(-1).astype(jnp.int32)
    wp = jnp.pad(weight, ((0, 0), (0, PDIM - DIM)))
    out = _gather(xf, wp)
    return out[:, :DIM].reshape(B, S, DIM)
